# baseline (speedup 1.0000x reference)
import jax
import jax.numpy as jnp
from jax.experimental import pallas as pl
from jax.experimental.pallas import tpu as pltpu

B = 16384
CTX = 128
D = 128
H = 128
K = 8
RH = 64
ROWS = 2048


def _fused_kernel(ctx_ref, m0, m1, m2, m3, m4, m5, m6, m7, gt_ref,
                  rw1_ref, rb1_ref, lng_ref, lnb_ref, rw2_ref, rb2_ref,
                  rw3_ref, rb3_ref, prior_ref, encw_ref, encb_ref, fw_ref,
                  fused_ref, sel_ref, probs_ref):
    f32 = jnp.float32
    dn_lane_lane = (((1,), (1,)), ((), ()))
    dn_lane_sub = (((1,), (0,)), ((), ()))

    ht = jax.lax.dot_general(rw1_ref[...], ctx_ref[...], dn_lane_lane,
                             preferred_element_type=f32) + rb1_ref[...]
    mu = jnp.mean(ht, axis=0, keepdims=True)
    var = jnp.mean((ht - mu) ** 2, axis=0, keepdims=True)
    ht = (ht - mu) / jnp.sqrt(var + 1e-5) * lng_ref[...] + lnb_ref[...]
    ht = jax.nn.relu(ht)
    h2t = jax.nn.relu(
        jax.lax.dot_general(rw2_ref[...], ht, dn_lane_sub,
                            preferred_element_type=f32) + rb2_ref[...])
    logits = jax.lax.dot_general(rw3_ref[...], h2t, dn_lane_sub,
                                 preferred_element_type=f32) + rb3_ref[...]
    logits = logits + prior_ref[...]
    probs_t = jax.nn.sigmoid(logits)
    u_t = gt_ref[...].T
    g_t = -jnp.log(-jnp.log(u_t + 1e-8) + 1e-8)
    sel_t = jax.nn.sigmoid(logits + g_t)

    iota = jax.lax.broadcasted_iota(jnp.int32, (K, ROWS), 0)
    m1v = jnp.max(probs_t, axis=0, keepdims=True)
    i1 = jnp.min(jnp.where(probs_t == m1v, iota, K), axis=0, keepdims=True)
    p2 = jnp.where(iota == i1, -jnp.inf, probs_t)
    m2v = jnp.max(p2, axis=0, keepdims=True)
    i2 = jnp.min(jnp.where(p2 == m2v, iota, K), axis=0, keepdims=True)
    minmask = (iota == i1) | (iota == i2)
    sel_t = jnp.maximum(sel_t, minmask.astype(f32))

    w = jax.nn.softmax(fw_ref[...], axis=0)
    coef_t = jnp.where(sel_t > 0.5, sel_t, 0.0) * w

    mods = (m0, m1, m2, m3, m4, m5, m6, m7)
    acc_t = None
    for k in range(K):
        enc_t = jax.lax.dot_general(encw_ref[k], mods[k][...], dn_lane_lane,
                                    preferred_element_type=f32)
        term = coef_t[k:k + 1, :] * enc_t
        acc_t = term if acc_t is None else acc_t + term

    bias = jnp.dot(coef_t.T, encb_ref[...], preferred_element_type=f32)
    fused_ref[...] = acc_t.T + bias
    sel_ref[...] = sel_t.T
    probs_ref[...] = probs_t.T


@jax.jit
def kernel(context, mod_0, mod_1, mod_2, mod_3, mod_4, mod_5, mod_6, mod_7,
           r_w1, r_b1, ln_g, ln_b, r_w2, r_b2, r_w3, r_b3, prior, enc_W,
           enc_b, fusion_w):
    f32 = jnp.float32
    u = jax.random.uniform(jax.random.key(1234), (B, K), dtype=f32)

    row = lambda shape: pl.BlockSpec(shape, lambda i: (i, 0))
    colblk = lambda shape: pl.BlockSpec(shape, lambda i: (0, i))
    full2 = lambda shape: pl.BlockSpec(shape, lambda i: (0, 0))

    grid = B // ROWS
    out_shapes = (
        jax.ShapeDtypeStruct((B, H), f32),
        jax.ShapeDtypeStruct((B, K), f32),
        jax.ShapeDtypeStruct((B, K), f32),
    )
    in_specs = (
        [row((ROWS, CTX))] + [row((ROWS, D))] * K + [row((ROWS, K))] +
        [full2((RH, CTX)), full2((RH, 1)), full2((RH, 1)), full2((RH, 1)),
         full2((RH // 2, RH)), full2((RH // 2, 1)), full2((K, RH // 2)),
         full2((K, 1)), full2((K, 1)),
         pl.BlockSpec((K, H, D), lambda i: (0, 0, 0)), full2((K, H)),
         full2((K, 1))]
    )
    out_specs = (row((ROWS, H)), row((ROWS, K)), row((ROWS, K)))

    fused, sel, probs = pl.pallas_call(
        _fused_kernel,
        grid=(grid,),
        in_specs=in_specs,
        out_specs=out_specs,
        out_shape=out_shapes,
        compiler_params=pltpu.CompilerParams(
            dimension_semantics=("parallel",)),
    )(context, mod_0, mod_1, mod_2, mod_3, mod_4, mod_5, mod_6, mod_7, u,
      r_w1, r_b1.reshape(RH, 1), ln_g.reshape(RH, 1), ln_b.reshape(RH, 1),
      r_w2, r_b2.reshape(RH // 2, 1), r_w3, r_b3.reshape(K, 1),
      prior.reshape(K, 1), enc_W, enc_b, fusion_w.reshape(K, 1))
    return fused, sel, probs


# speedup vs baseline: 1.5504x; 1.5504x over previous
import jax
import jax.numpy as jnp
from jax.experimental import pallas as pl
from jax.experimental.pallas import tpu as pltpu

B = 16384
CTX = 128
D = 128
H = 128
K = 8
RH = 64
ROWS = 2048


def _fused_kernel(ctx_ref, m0, m1, m2, m3, m4, m5, m6, m7, gt_ref,
                  rw1_ref, rb1_ref, lng_ref, lnb_ref, rw2_ref, rb2_ref,
                  rw3_ref, rb3_ref, prior_ref, encw_ref, encb_ref, fw_ref,
                  fused_ref, sel_ref, probs_ref):
    f32 = jnp.float32
    dn_lane_lane = (((1,), (1,)), ((), ()))
    dn_lane_sub = (((1,), (0,)), ((), ()))

    ht = jax.lax.dot_general(rw1_ref[...], ctx_ref[...], dn_lane_lane,
                             preferred_element_type=f32) + rb1_ref[...]
    mu = jnp.mean(ht, axis=0, keepdims=True)
    var = jnp.mean((ht - mu) ** 2, axis=0, keepdims=True)
    ht = (ht - mu) / jnp.sqrt(var + 1e-5) * lng_ref[...] + lnb_ref[...]
    ht = jax.nn.relu(ht)
    h2t = jax.nn.relu(
        jax.lax.dot_general(rw2_ref[...], ht, dn_lane_sub,
                            preferred_element_type=f32) + rb2_ref[...])
    logits = jax.lax.dot_general(rw3_ref[...], h2t, dn_lane_sub,
                                 preferred_element_type=f32) + rb3_ref[...]
    logits = logits + prior_ref[...]
    probs_t = jax.nn.sigmoid(logits)
    sel_t = jax.nn.sigmoid(logits + gt_ref[...])

    iota = jax.lax.broadcasted_iota(jnp.int32, (K, ROWS), 0)
    m1v = jnp.max(probs_t, axis=0, keepdims=True)
    i1 = jnp.min(jnp.where(probs_t == m1v, iota, K), axis=0, keepdims=True)
    p2 = jnp.where(iota == i1, -jnp.inf, probs_t)
    m2v = jnp.max(p2, axis=0, keepdims=True)
    i2 = jnp.min(jnp.where(p2 == m2v, iota, K), axis=0, keepdims=True)
    minmask = (iota == i1) | (iota == i2)
    sel_t = jnp.maximum(sel_t, minmask.astype(f32))

    w = jax.nn.softmax(fw_ref[...], axis=0)
    coef_t = jnp.where(sel_t > 0.5, sel_t, 0.0) * w

    mods = (m0, m1, m2, m3, m4, m5, m6, m7)
    acc_t = None
    for k in range(K):
        enc_t = jax.lax.dot_general(encw_ref[k].astype(jnp.bfloat16),
                                    mods[k][...].astype(jnp.bfloat16),
                                    dn_lane_lane,
                                    preferred_element_type=f32)
        term = coef_t[k:k + 1, :] * enc_t
        acc_t = term if acc_t is None else acc_t + term

    bias = jnp.dot(coef_t.T, encb_ref[...], preferred_element_type=f32)
    fused_ref[...] = acc_t.T + bias
    sel_ref[...] = sel_t.T
    probs_ref[...] = probs_t.T


@jax.jit
def kernel(context, mod_0, mod_1, mod_2, mod_3, mod_4, mod_5, mod_6, mod_7,
           r_w1, r_b1, ln_g, ln_b, r_w2, r_b2, r_w3, r_b3, prior, enc_W,
           enc_b, fusion_w):
    f32 = jnp.float32
    u = jax.random.uniform(jax.random.key(1234), (B, K), dtype=f32)
    g_t = (-jnp.log(-jnp.log(u + 1e-8) + 1e-8)).T

    row = lambda shape: pl.BlockSpec(shape, lambda i: (i, 0))
    colblk = lambda shape: pl.BlockSpec(shape, lambda i: (0, i))
    full2 = lambda shape: pl.BlockSpec(shape, lambda i: (0, 0))

    grid = B // ROWS
    out_shapes = (
        jax.ShapeDtypeStruct((B, H), f32),
        jax.ShapeDtypeStruct((B, K), f32),
        jax.ShapeDtypeStruct((B, K), f32),
    )
    in_specs = (
        [row((ROWS, CTX))] + [row((ROWS, D))] * K + [colblk((K, ROWS))] +
        [full2((RH, CTX)), full2((RH, 1)), full2((RH, 1)), full2((RH, 1)),
         full2((RH // 2, RH)), full2((RH // 2, 1)), full2((K, RH // 2)),
         full2((K, 1)), full2((K, 1)),
         pl.BlockSpec((K, H, D), lambda i: (0, 0, 0)), full2((K, H)),
         full2((K, 1))]
    )
    out_specs = (row((ROWS, H)), row((ROWS, K)), row((ROWS, K)))

    fused, sel, probs = pl.pallas_call(
        _fused_kernel,
        grid=(grid,),
        in_specs=in_specs,
        out_specs=out_specs,
        out_shape=out_shapes,
        compiler_params=pltpu.CompilerParams(
            dimension_semantics=("parallel",)),
    )(context, mod_0, mod_1, mod_2, mod_3, mod_4, mod_5, mod_6, mod_7, g_t,
      r_w1, r_b1.reshape(RH, 1), ln_g.reshape(RH, 1), ln_b.reshape(RH, 1),
      r_w2, r_b2.reshape(RH // 2, 1), r_w3, r_b3.reshape(K, 1),
      prior.reshape(K, 1), enc_W, enc_b, fusion_w.reshape(K, 1))
    return fused, sel, probs
